# baseline (speedup 1.0000x reference)
import functools

import jax
import jax.numpy as jnp
from jax import lax
from jax.experimental import pallas as pl
from jax.experimental.pallas import tpu as pltpu
from jax.experimental.pallas import tpu_sc as plsc

N_ROWS = 100000
EMB = 128
OUT_D = 480
ZPAD = OUT_D - EMB

NW = 32
CHUNK = 80
CHP = 40
SLAB = CHUNK * CHP


def _t2_body(tp_ref, w_ref, o_ref):
    mm = jnp.dot(tp_ref[...], w_ref[...], preferred_element_type=jnp.float32)
    o_ref[...] = mm * (1.0 / (EMB ** 0.5))


_t2_call = pl.pallas_call(
    _t2_body,
    out_shape=jax.ShapeDtypeStruct((EMB, EMB), jnp.float32),
)


@functools.cache
def _make_sc_fill():
    @functools.partial(
        pl.kernel,
        out_type=jax.ShapeDtypeStruct((N_ROWS, OUT_D), jnp.float32),
        mesh=plsc.VectorSubcoreMesh(core_axis_name="c", subcore_axis_name="s"),
        scratch_types=[
            pltpu.VMEM((SLAB,), jnp.int32),
            pltpu.VMEM((2, CHUNK, EMB), jnp.float32),
            pltpu.VMEM((2, CHUNK, OUT_D), jnp.float32),
            pltpu.VMEM((EMB, EMB), jnp.float32),
            pltpu.VMEM_SHARED((EMB, EMB), jnp.float32),
            pltpu.SemaphoreType.DMA((2,)),
            pltpu.SemaphoreType.DMA((2,)),
        ],
    )
    def _sc_fill(t2_hbm, z_hbm, out_hbm, idx_v, rows_v, buf_v, t2_v, t2_sh,
                 gsem, wsem):
        wid = lax.axis_index("s") * 2 + lax.axis_index("c")
        bw = jnp.minimum(wid * SLAB, N_ROWS - SLAB)

        @pl.when(lax.axis_index("s") == 0)
        def _():
            pltpu.sync_copy(t2_hbm, t2_v)
            pltpu.sync_copy(t2_v, t2_sh)
        plsc.subcore_barrier()

        for b in range(2):
            def zrow(r, carry):
                def zcol(c2, carry2):
                    buf_v[b, r, pl.ds(EMB + c2 * 16, 16)] = (
                        jnp.zeros((16,), jnp.float32))
                    return carry2
                return lax.fori_loop(0, ZPAD // 16, zcol, carry)
            lax.fori_loop(0, CHUNK, zrow, 0)

        def g_copy(k, b):
            return pltpu.make_async_copy(
                t2_sh.at[idx_v.at[pl.ds(k * CHUNK, CHUNK)]],
                rows_v.at[b], gsem.at[b])

        def c_vec(b):
            def crow(r, carry):
                for c in range(EMB // 16):
                    buf_v[b, r, pl.ds(c * 16, 16)] = (
                        rows_v[b, r, pl.ds(c * 16, 16)])
                return carry
            lax.fori_loop(0, CHUNK, crow, 0)

        def w_copy(k, b):
            return pltpu.make_async_copy(
                buf_v.at[b],
                out_hbm.at[pl.ds(bw + k * CHUNK, CHUNK)],
                wsem.at[b])

        pltpu.sync_copy(z_hbm.at[pl.ds(bw, SLAB)], idx_v)

        g_copy(0, 0).start()
        g_copy(0, 0).wait()
        g_copy(1, 1).start()
        c_vec(0)
        w_copy(0, 0).start()
        g_copy(1, 1).wait()
        g_copy(2, 0).start()
        c_vec(1)
        w_copy(1, 1).start()

        def body(j, carry):
            for o in range(2):
                k = 2 * j + 2 + o
                b = o
                w_copy(k - 2, b).wait()
                g_copy(k, b).wait()

                @pl.when(k + 1 < CHP)
                def _():
                    g_copy(k + 1, 1 - b).start()

                c_vec(b)
                w_copy(k, b).start()
            return carry
        lax.fori_loop(0, (CHP - 2) // 2, body, 0)

        w_copy(CHP - 2, 0).wait()
        w_copy(CHP - 1, 1).wait()

    return _sc_fill


def kernel(z, table, W):
    tp = jnp.pad(table, ((0, EMB - table.shape[0]), (0, 0)))
    t2 = _t2_call(tp, W)
    return _make_sc_fill()(t2, z.astype(jnp.int32))


# speedup vs baseline: 1.1197x; 1.1197x over previous
import functools

import jax
import jax.numpy as jnp
from jax import lax
from jax.experimental import pallas as pl
from jax.experimental.pallas import tpu as pltpu
from jax.experimental.pallas import tpu_sc as plsc

N_ROWS = 100000
EMB = 128
OUT_D = 480
ZPAD = OUT_D - EMB

NW = 32
CHUNK = 128
CHP = 25
SLAB = CHUNK * CHP
NBUF = 4


def _t2_body(tp_ref, w_ref, o_ref):
    mm = jnp.dot(tp_ref[...], w_ref[...], preferred_element_type=jnp.float32)
    o_ref[...] = mm * (1.0 / (EMB ** 0.5))


_t2_call = pl.pallas_call(
    _t2_body,
    out_shape=jax.ShapeDtypeStruct((EMB, EMB), jnp.float32),
)


@functools.cache
def _make_sc_gather():
    @functools.partial(
        pl.kernel,
        out_type=jax.ShapeDtypeStruct((N_ROWS, OUT_D), jnp.float32),
        mesh=plsc.VectorSubcoreMesh(core_axis_name="c", subcore_axis_name="s"),
        scratch_types=[
            pltpu.VMEM((SLAB,), jnp.int32),
            pltpu.VMEM((NBUF, CHUNK, EMB), jnp.float32),
            pltpu.VMEM((CHUNK, ZPAD), jnp.float32),
            pltpu.VMEM_SHARED((EMB, EMB), jnp.float32),
            pltpu.SemaphoreType.DMA((NBUF,)),
            pltpu.SemaphoreType.DMA((NBUF,)),
            pltpu.SemaphoreType.DMA,
            pltpu.SemaphoreType.DMA,
        ],
    )
    def _sc_gather(t2_hbm, z_hbm, out_hbm, idx_v, rows_v, zbuf_v, t2_sh,
                   gsem, wsem, zsem, isem):
        wid = lax.axis_index("s") * 2 + lax.axis_index("c")
        bw = jnp.minimum(wid * SLAB, N_ROWS - SLAB)

        @pl.when(lax.axis_index("s") == 0)
        def _():
            pltpu.sync_copy(t2_hbm, rows_v.at[0])
            pltpu.sync_copy(rows_v.at[0], t2_sh)

        i_desc = pltpu.make_async_copy(z_hbm.at[pl.ds(bw, SLAB)], idx_v, isem)
        i_desc.start()

        def zrow(r, carry):
            def zcol(c2, carry2):
                zbuf_v[r, pl.ds(c2 * 16, 16)] = jnp.zeros((16,), jnp.float32)
                return carry2
            return lax.fori_loop(0, ZPAD // 16, zcol, carry)
        lax.fori_loop(0, CHUNK, zrow, 0)

        def g_copy(k, b):
            return pltpu.make_async_copy(
                t2_sh.at[idx_v.at[pl.ds(k * CHUNK, CHUNK)]],
                rows_v.at[b], gsem.at[b])

        def w_copy(k, b):
            return pltpu.make_async_copy(
                rows_v.at[b],
                out_hbm.at[pl.ds(bw + k * CHUNK, CHUNK), pl.ds(0, EMB)],
                wsem.at[b])

        def z_copy(k):
            return pltpu.make_async_copy(
                zbuf_v,
                out_hbm.at[pl.ds(bw + k * CHUNK, CHUNK), pl.ds(EMB, ZPAD)],
                zsem)

        for k in range(CHP):
            z_copy(k).start()

        i_desc.wait()
        plsc.subcore_barrier()

        for b in range(NBUF):
            g_copy(b, b).start()

        def tail(k, b):
            g_copy(k, b).wait()
            w_copy(k, b).start()

        for k in range(NBUF - 1):
            tail(k, k)

        def body(j, carry):
            for o in range(NBUF):
                k = NBUF * j + NBUF + o
                b = o
                w_copy(k - NBUF, b).wait()
                g_copy(k, b).start()
                tail(k - 1, (o + NBUF - 1) % NBUF)
            return carry
        lax.fori_loop(0, (CHP - NBUF - 1) // NBUF, body, 0)

        w_copy(CHP - 1 - NBUF, 0).wait()
        g_copy(CHP - 1, 0).start()
        tail(CHP - 2, (CHP - 2) % NBUF)
        tail(CHP - 1, (CHP - 1) % NBUF)
        for k in range(CHP - NBUF, CHP):
            w_copy(k, k % NBUF).wait()
        for k in range(CHP):
            z_copy(k).wait()

    return _sc_gather


def kernel(z, table, W):
    tp = jnp.pad(table, ((0, EMB - table.shape[0]), (0, 0)))
    t2 = _t2_call(tp, W)
    return _make_sc_gather()(t2, z.astype(jnp.int32))
